# baseline (speedup 1.0000x reference)
import jax
import jax.numpy as jnp
from jax import lax
from jax.experimental import pallas as pl
from jax.experimental.pallas import tpu as pltpu
from jax.experimental.pallas import tpu_sc as plsc

_BATCH = 16384
_N_ROWS = 1000000
_D = 64
_LANES = 16
_NC = 2
_NS = 16
_NW = _NC * _NS
_BPW = _BATCH // _NW
_CHUNK = 128
_NCHUNK = _BPW // _CHUNK
_LAG = 4
_Y_HIGH = 5.5


def _body(x_hbm, uf_hbm, ub_hbm, mf_hbm, mb_hbm, out_hbm,
          uidx, midx, utr, mtr, ubias, mbias, res, sem):
    wid = lax.axis_index("s") * _NC + lax.axis_index("c")
    base = wid * _BPW
    for ch in range(_NCHUNK):
        pltpu.sync_copy(x_hbm.at[pl.ds(base + ch * _CHUNK, _CHUNK)],
                        uidx.at[ch])
        pltpu.sync_copy(x_hbm.at[pl.ds(_BATCH + base + ch * _CHUNK, _CHUNK)],
                        midx.at[ch])

    def wait8(_):
        for _k in range(8):
            pltpu.make_async_copy(ub_hbm.at[uidx.at[0]],
                                  ubias.at[pl.ds(0, _CHUNK)], sem).wait()

    for ch in range(_NCHUNK):
        pltpu.async_copy(ub_hbm.at[uidx.at[ch]],
                         ubias.at[pl.ds(ch * _CHUNK, _CHUNK)], sem)
        pltpu.async_copy(mb_hbm.at[midx.at[ch]],
                         mbias.at[pl.ds(ch * _CHUNK, _CHUNK)], sem)

    def fire(f, carry):
        fbase = f * _N_ROWS
        for ch in range(_NCHUNK):
            sl = pl.ds(ch * _CHUNK, _CHUNK)
            pltpu.async_copy(
                uf_hbm.at[pl.ds(fbase, _N_ROWS)].at[uidx.at[ch]],
                utr.at[f, sl], sem)
            pltpu.async_copy(
                mf_hbm.at[pl.ds(fbase, _N_ROWS)].at[midx.at[ch]],
                mtr.at[f, sl], sem)

        @pl.when(f >= _LAG)
        def _():
            wait8(None)

        return carry

    lax.fori_loop(0, _D, fire, 0)
    for _i in range(_LAG):
        wait8(None)
    wait8(None)

    def group_body(g, carry):
        sl = pl.ds(g * _LANES, _LANES)
        acc = ubias[sl] + mbias[sl]
        for f in range(_D):
            acc = acc + utr[f, sl] * mtr[f, sl]
        res[sl] = _Y_HIGH / (1.0 + jnp.exp(-acc))
        return carry

    lax.fori_loop(0, _BPW // _LANES, group_body, 0)
    pltpu.sync_copy(res, out_hbm.at[pl.ds(base, _BPW)])


@jax.jit
def kernel(x, user_factors, user_bias, movie_factors, movie_bias):
    f = pl.kernel(
        _body,
        out_type=jax.ShapeDtypeStruct((_BATCH,), jnp.float32),
        mesh=plsc.VectorSubcoreMesh(core_axis_name="c", subcore_axis_name="s"),
        compiler_params=pltpu.CompilerParams(needs_layout_passes=False),
        scratch_types=[
            pltpu.VMEM((_NCHUNK, _CHUNK), jnp.int32),
            pltpu.VMEM((_NCHUNK, _CHUNK), jnp.int32),
            pltpu.VMEM((_D, _BPW), jnp.float32),
            pltpu.VMEM((_D, _BPW), jnp.float32),
            pltpu.VMEM((_BPW,), jnp.float32),
            pltpu.VMEM((_BPW,), jnp.float32),
            pltpu.VMEM((_BPW,), jnp.float32),
            pltpu.SemaphoreType.DMA,
        ],
    )
    out = f(x.T.reshape(2 * _BATCH),
            user_factors.T.reshape(_D * _N_ROWS),
            user_bias.T.reshape(_N_ROWS),
            movie_factors.T.reshape(_D * _N_ROWS),
            movie_bias.T.reshape(_N_ROWS))
    return out.reshape(_BATCH, 1)
